# baseline (speedup 1.0000x reference)
import jax
import jax.numpy as jnp
from jax import lax
from jax.experimental import pallas as pl
from jax.experimental.pallas import tpu as pltpu
from jax.experimental.pallas import tpu_sc as plsc

_HIDDEN = 4096
_E = 64
_TOPK = 8
_BT = 1024
_NC = 2
_NS = 16
_NW = _NC * _NS
_C = 128
_L = 16


def _matmul_body(w_ref, b_ref, x_ref, out_ref):
    out_ref[...] = jax.lax.dot_general(
        w_ref[...], x_ref[...], (((1,), (1,)), ((), ())),
        preferred_element_type=jnp.float32,
        precision=jax.lax.Precision.DEFAULT) + b_ref[...]


def _matmul_call(x, W, b2):
    T = x.shape[0]
    return pl.pallas_call(
        _matmul_body,
        grid=(T // _BT,),
        in_specs=[
            pl.BlockSpec((_E, _HIDDEN), lambda i: (0, 0)),
            pl.BlockSpec((_E, 1), lambda i: (0, 0)),
            pl.BlockSpec((_BT, _HIDDEN), lambda i: (i, 0)),
        ],
        out_specs=pl.BlockSpec((_E, _BT), lambda i: (0, i)),
        out_shape=jax.ShapeDtypeStruct((_E, T), jnp.float32),
    )(W, b2, x)


def _sc_route_body(logits_hbm, wts_hbm, sel_hbm, mask_hbm,
                   lg_v, wb_v, sb_v, mb_v):
    T = logits_hbm.shape[1]
    tw = T // _NW
    wid = lax.axis_index("s") * _NC + lax.axis_index("c")
    base0 = wid * tw

    def chunk_step(c, _):
        base = base0 + c * _C
        pltpu.sync_copy(logits_hbm.at[:, pl.ds(base, _C)], lg_v)

        def group(g, _):
            t0 = g * _L
            topv = [jnp.full((_L,), -jnp.inf, jnp.float32)
                    for _j in range(_TOPK)]
            topi = [jnp.zeros((_L,), jnp.int32) for _j in range(_TOPK)]
            for e in range(_E):
                v = lg_v[e, pl.ds(t0, _L)]
                ei = jnp.full((_L,), e, jnp.int32)
                for j in range(_TOPK):
                    gt = v > topv[j]
                    nv = jnp.where(gt, v, topv[j])
                    ni = jnp.where(gt, ei, topi[j])
                    v = jnp.where(gt, topv[j], v)
                    ei = jnp.where(gt, topi[j], ei)
                    topv[j] = nv
                    topi[j] = ni
            m = topv[0]
            exps = [jnp.exp(tv - m) for tv in topv]
            s = exps[0]
            for j in range(1, _TOPK):
                s = s + exps[j]
            rs = 1.0 / s
            for j in range(_TOPK):
                wb_v[j, pl.ds(t0, _L)] = exps[j] * rs
                sb_v[j, pl.ds(t0, _L)] = topi[j]
            onev = jnp.ones((_L,), jnp.int32)
            zerov = jnp.zeros((_L,), jnp.int32)
            for e in range(_E):
                for j in range(_TOPK):
                    mb_v[e * _TOPK + j, pl.ds(t0, _L)] = jnp.where(
                        topi[j] == e, onev, zerov)
            return 0

        lax.fori_loop(0, _C // _L, group, 0)
        pltpu.sync_copy(wb_v, wts_hbm.at[:, pl.ds(base, _C)])
        pltpu.sync_copy(sb_v, sel_hbm.at[:, pl.ds(base, _C)])
        pltpu.sync_copy(mb_v, mask_hbm.at[:, pl.ds(base, _C)])
        return 0

    lax.fori_loop(0, tw // _C, chunk_step, 0)


def _sc_route(logitsT):
    T = logitsT.shape[1]
    mesh = plsc.VectorSubcoreMesh(core_axis_name="c", subcore_axis_name="s")
    return pl.kernel(
        _sc_route_body,
        out_type=[
            jax.ShapeDtypeStruct((_TOPK, T), jnp.float32),
            jax.ShapeDtypeStruct((_TOPK, T), jnp.int32),
            jax.ShapeDtypeStruct((_E * _TOPK, T), jnp.int32),
        ],
        mesh=mesh,
        scratch_types=[
            pltpu.VMEM((_E, _C), jnp.float32),
            pltpu.VMEM((_TOPK, _C), jnp.float32),
            pltpu.VMEM((_TOPK, _C), jnp.int32),
            pltpu.VMEM((_E * _TOPK, _C), jnp.int32),
        ],
    )(logitsT)


@jax.jit
def kernel(x, W, b):
    T = x.shape[0]
    logitsT = _matmul_call(x, W, b.reshape(_E, 1))
    wtsT, selT, mask2d = _sc_route(logitsT)
    return (logitsT.T, wtsT.T, selT.T,
            mask2d.reshape(_E, _TOPK, T))


# speedup vs baseline: 1.4771x; 1.4771x over previous
import jax
import jax.numpy as jnp
from jax.experimental import pallas as pl

_HIDDEN = 4096
_E = 64
_TOPK = 8
_BT = 1024


def _router_body(w_ref, b_ref, x_ref, logits_ref, wts_ref, sel_ref, mask_ref):
    w = w_ref[...]
    x = x_ref[...]
    logits = jax.lax.dot_general(
        w, x, (((1,), (1,)), ((), ())),
        preferred_element_type=jnp.float32,
        precision=jax.lax.Precision.DEFAULT)
    logits = logits + b_ref[...]
    logits_ref[...] = logits

    eio = jax.lax.broadcasted_iota(jnp.int32, (_E, _BT), 0)
    work = logits
    neg_inf = jnp.float32(-jnp.inf)
    vals = []
    idxs = []
    for k in range(_TOPK):
        m = jnp.max(work, axis=0, keepdims=True)
        ismax = work == m
        idx = jnp.min(jnp.where(ismax, eio, _E), axis=0, keepdims=True)
        onehot = eio == idx
        mask_ref[:, k, :] = onehot.astype(jnp.int32)
        vals.append(m)
        idxs.append(idx)
        work = jnp.where(onehot, neg_inf, work)

    vals = jnp.concatenate(vals, axis=0)
    sel = jnp.concatenate(idxs, axis=0)
    e = jnp.exp(vals - vals[0:1])
    wts_ref[...] = e / jnp.sum(e, axis=0, keepdims=True)
    sel_ref[...] = sel


def _router_call(x, W, b2, interpret=False):
    T = x.shape[0]
    return pl.pallas_call(
        _router_body,
        grid=(T // _BT,),
        in_specs=[
            pl.BlockSpec((_E, _HIDDEN), lambda i: (0, 0)),
            pl.BlockSpec((_E, 1), lambda i: (0, 0)),
            pl.BlockSpec((_BT, _HIDDEN), lambda i: (i, 0)),
        ],
        out_specs=[
            pl.BlockSpec((_E, _BT), lambda i: (0, i)),
            pl.BlockSpec((_TOPK, _BT), lambda i: (0, i)),
            pl.BlockSpec((_TOPK, _BT), lambda i: (0, i)),
            pl.BlockSpec((_E, _TOPK, _BT), lambda i: (0, 0, i)),
        ],
        out_shape=[
            jax.ShapeDtypeStruct((_E, T), jnp.float32),
            jax.ShapeDtypeStruct((_TOPK, T), jnp.float32),
            jax.ShapeDtypeStruct((_TOPK, T), jnp.int32),
            jax.ShapeDtypeStruct((_E, _TOPK, T), jnp.int32),
        ],
        interpret=interpret,
    )(W, b2, x)


@jax.jit
def kernel(x, W, b):
    logitsT, wtsT, selT, mask = _router_call(x, W, b.reshape(_E, 1))
    return (logitsT.T, wtsT.T, selT.T, mask)
